# Initial kernel scaffold; baseline (speedup 1.0000x reference)
import functools

import jax
import jax.numpy as jnp
from jax import lax
from jax.experimental import pallas as pl
from jax.experimental.pallas import tpu as pltpu
from jax.experimental.pallas import tpu_sc as plsc

_NC = 2
_NS = 16
_LANES = 16
_NW = _NC * _NS

_INT32_MAX = jnp.int32(2147483647)


def _scores_body(x_ref, w1_ref, b1_ref, w2t_ref, b2_ref, e_ref):
    xb = x_ref[...]
    t = jnp.tanh(
        jnp.dot(xb, w1_ref[...], preferred_element_type=jnp.float32)
        + b1_ref[...]
    )
    s = jnp.sum(t * w2t_ref[...], axis=1, keepdims=True) + b2_ref[...]
    r = xb.shape[0]
    e_ref[...] = jnp.exp(s).reshape(1, 1, r)


def _scores_call(x, W1, b1, W2, b2, block_rows):
    n, d = x.shape
    h = W1.shape[1]
    nb = n // block_rows
    return pl.pallas_call(
        _scores_body,
        grid=(nb,),
        in_specs=[
            pl.BlockSpec((block_rows, d), lambda k: (k, 0)),
            pl.BlockSpec((d, h), lambda k: (0, 0)),
            pl.BlockSpec((1, h), lambda k: (0, 0)),
            pl.BlockSpec((1, h), lambda k: (0, 0)),
            pl.BlockSpec((1, 1), lambda k: (0, 0)),
        ],
        out_specs=pl.BlockSpec((1, 1, block_rows), lambda k: (k, 0, 0)),
        out_shape=jax.ShapeDtypeStruct((nb, 1, block_rows), jnp.float32),
    )(x, W1, b1.reshape(1, h), W2.reshape(1, h), b2.reshape(1, 1)).reshape(n)


def _sc_pool_call(x, e, person_ids, p, chunk):
    n, d = x.shape
    rows_per_w = n // _NW
    nchunks = rows_per_w // chunk
    ncc = d // _LANES
    zp = (p // _NS) // 8 * 8
    tail = p - _NS * zp

    mesh = plsc.VectorSubcoreMesh(
        core_axis_name="c", subcore_axis_name="s",
        num_cores=_NC, num_subcores=_NS,
    )

    @functools.partial(
        pl.kernel,
        out_type=[
            jax.ShapeDtypeStruct((_NC * p, d), jnp.float32),
            jax.ShapeDtypeStruct((_NC * p, _LANES), jnp.float32),
        ],
        mesh=mesh,
        scratch_types=[
            pltpu.VMEM_SHARED((p, d), jnp.float32),
            pltpu.VMEM_SHARED((p, _LANES), jnp.float32),
            pltpu.VMEM((chunk, d), jnp.float32),
            pltpu.VMEM((chunk, d), jnp.float32),
            pltpu.VMEM((chunk, _LANES), jnp.float32),
            pltpu.VMEM((chunk,), jnp.float32),
            pltpu.VMEM((chunk,), jnp.int32),
        ],
    )
    def sc_kernel(x_hbm, e_hbm, pid_hbm, outx_hbm, oute_hbm,
                  accx, acce, xbuf, sbuf, etile, ebuf, pidbuf):
        c = lax.axis_index("c")
        s = lax.axis_index("s")
        wid = s * _NC + c
        zero = jnp.zeros((_LANES,), jnp.float32)

        @pl.loop(0, chunk)
        def _zrow(i):
            for cc in range(ncc):
                sbuf[i, pl.ds(cc * _LANES, _LANES)] = zero
            etile[i, :] = zero

        nfull = zp // chunk
        for k in range(nfull):
            r0 = s * zp + k * chunk
            pltpu.sync_copy(sbuf, accx.at[pl.ds(r0, chunk)])
            pltpu.sync_copy(etile, acce.at[pl.ds(r0, chunk)])
        rem = zp - nfull * chunk
        if rem:
            r0 = s * zp + nfull * chunk
            pltpu.sync_copy(sbuf.at[pl.ds(0, rem)], accx.at[pl.ds(r0, rem)])
            pltpu.sync_copy(etile.at[pl.ds(0, rem)], acce.at[pl.ds(r0, rem)])

        if tail:
            @pl.when(s == 0)
            def _ztail():
                pltpu.sync_copy(sbuf.at[pl.ds(0, tail)],
                                accx.at[pl.ds(_NS * zp, tail)])
                pltpu.sync_copy(etile.at[pl.ds(0, tail)],
                                acce.at[pl.ds(_NS * zp, tail)])

        plsc.subcore_barrier()

        base0 = wid * rows_per_w
        lane_iota = lax.iota(jnp.int32, _LANES)

        @pl.loop(0, nchunks)
        def _chunk(j):
            base = base0 + j * chunk
            pltpu.sync_copy(e_hbm.at[pl.ds(base, chunk)], ebuf)
            pltpu.sync_copy(pid_hbm.at[pl.ds(base, chunk)], pidbuf)
            pltpu.sync_copy(x_hbm.at[pl.ds(base, chunk)], xbuf)

            @pl.loop(0, chunk)
            def _row(i):
                eb = plsc.load_gather(
                    ebuf, [jnp.zeros((_LANES,), jnp.int32) + i])
                for cc in range(ncc):
                    sl = pl.ds(cc * _LANES, _LANES)
                    sbuf[i, sl] = xbuf[i, sl] * eb
                etile[i, :] = jnp.where(lane_iota == 0, eb, 0.0)

            pltpu.sync_copy(sbuf, accx.at[pidbuf], add=True)
            pltpu.sync_copy(etile, acce.at[pidbuf], add=True)

        plsc.subcore_barrier()

        out0 = c * p
        pltpu.sync_copy(accx.at[pl.ds(s * zp, zp)],
                        outx_hbm.at[pl.ds(out0 + s * zp, zp)])
        pltpu.sync_copy(acce.at[pl.ds(s * zp, zp)],
                        oute_hbm.at[pl.ds(out0 + s * zp, zp)])
        if tail:
            @pl.when(s == 0)
            def _ctail():
                pltpu.sync_copy(accx.at[pl.ds(_NS * zp, tail)],
                                outx_hbm.at[pl.ds(out0 + _NS * zp, tail)])
                pltpu.sync_copy(acce.at[pl.ds(_NS * zp, tail)],
                                oute_hbm.at[pl.ds(out0 + _NS * zp, tail)])

    return sc_kernel(x, e, person_ids)


def _fin_body(ax_ref, ae_ref, pooled_ref, pid_ref):
    ax = ax_ref[...]
    ae = ae_ref[...]
    sx = ax[0] + ax[1]
    se16 = ae[0] + ae[1]
    se = se16[:, 0:1]
    nonempty = se > 0.0
    inv = jnp.where(nonempty, 1.0 / jnp.where(nonempty, se, 1.0), 0.0)
    pooled_ref[...] = sx * inv
    rp = sx.shape[0]
    k = pl.program_id(0)
    lanes = jax.lax.broadcasted_iota(jnp.int32, (1, 1, rp), 2) + k * rp
    pid_ref[...] = jnp.where(nonempty.reshape(1, 1, rp), lanes, _INT32_MAX)


def _fin_call(accx, acce, p, d, block_p):
    nbp = p // block_p
    return pl.pallas_call(
        _fin_body,
        grid=(nbp,),
        in_specs=[
            pl.BlockSpec((2, block_p, d), lambda k: (0, k, 0)),
            pl.BlockSpec((2, block_p, _LANES), lambda k: (0, k, 0)),
        ],
        out_specs=[
            pl.BlockSpec((block_p, d), lambda k: (k, 0)),
            pl.BlockSpec((1, 1, block_p), lambda k: (k, 0, 0)),
        ],
        out_shape=[
            jax.ShapeDtypeStruct((p, d), jnp.float32),
            jax.ShapeDtypeStruct((nbp, 1, block_p), jnp.int32),
        ],
    )(accx.reshape(2, p, d), acce.reshape(2, p, _LANES))


def kernel(x, W1, b1, W2, b2, person_ids):
    n, d = x.shape
    p = 10000
    e = _scores_call(x, W1, b1, W2, b2, block_rows=512)
    outx, oute = _sc_pool_call(x, e, person_ids, p, chunk=80)
    pooled, pids3 = _fin_call(outx, oute, p, d, block_p=2000)
    return pooled, pids3.reshape(p)


# speedup vs baseline: 3.9904x; 3.9904x over previous
import functools

import jax
import jax.numpy as jnp
from jax import lax
from jax.experimental import pallas as pl
from jax.experimental.pallas import tpu as pltpu
from jax.experimental.pallas import tpu_sc as plsc

_NC = 2
_NS = 16
_LANES = 16
_NW = _NC * _NS

_INT32_MAX = 2147483647


def _scores_body(x_ref, w1_ref, b1_ref, w2t_ref, b2_ref, e_ref):
    xb = x_ref[...]
    t = jnp.tanh(
        jnp.dot(xb, w1_ref[...], preferred_element_type=jnp.float32)
        + b1_ref[...]
    )
    s = jnp.sum(t * w2t_ref[...], axis=1, keepdims=True) + b2_ref[...]
    e = jnp.exp(s)
    r = e.shape[0]
    e8 = e.reshape(r // 8, 8, 1)
    e_ref[...] = jnp.broadcast_to(e8, (r // 8, 8, _LANES)).reshape(r // 8, 128)


def _scores_call(x, W1, b1, W2, b2, block_rows):
    n, d = x.shape
    h = W1.shape[1]
    nb = n // block_rows
    return pl.pallas_call(
        _scores_body,
        grid=(nb,),
        in_specs=[
            pl.BlockSpec((block_rows, d), lambda k: (k, 0)),
            pl.BlockSpec((d, h), lambda k: (0, 0)),
            pl.BlockSpec((1, h), lambda k: (0, 0)),
            pl.BlockSpec((1, h), lambda k: (0, 0)),
            pl.BlockSpec((1, 1), lambda k: (0, 0)),
        ],
        out_specs=pl.BlockSpec((block_rows // 8, 128), lambda k: (k, 0)),
        out_shape=jax.ShapeDtypeStruct((n // 8, 128), jnp.float32),
    )(x, W1, b1.reshape(1, h), W2.reshape(1, h), b2.reshape(1, 1))


def _sc_pool_call(x, e_packed, pid_packed, person_ids, p, p2, chunk):
    n, d = x.shape
    nchunks_total = n // chunk
    base_cnt = nchunks_total // _NW
    extra = nchunks_total - base_cnt * _NW
    ncc = d // _LANES
    zp = p2 // _NS

    mesh = plsc.VectorSubcoreMesh(
        core_axis_name="c", subcore_axis_name="s",
        num_cores=_NC, num_subcores=_NS,
    )

    @functools.partial(
        pl.kernel,
        out_type=jax.ShapeDtypeStruct((_NC * p2, d), jnp.float32),
        mesh=mesh,
        scratch_types=[
            pltpu.VMEM_SHARED((p2, d), jnp.float32),
            pltpu.VMEM((chunk, d), jnp.float32),
            pltpu.VMEM((chunk, d), jnp.float32),
            pltpu.VMEM((chunk, d), jnp.float32),
            pltpu.VMEM((chunk // 8, d), jnp.float32),
            pltpu.VMEM((chunk // 8, d), jnp.int32),
            pltpu.VMEM((chunk,), jnp.int32),
            pltpu.VMEM((chunk,), jnp.int32),
        ],
    )
    def sc_kernel(x_hbm, e_hbm, pk_hbm, pid_hbm, out_hbm,
                  acc, xbuf, sbuf, dbuf, ebuf, pkbuf, pidbuf, pid2buf):
        c = lax.axis_index("c")
        s = lax.axis_index("s")
        wid = s * _NC + c
        zero = jnp.zeros((_LANES,), jnp.float32)

        @pl.loop(0, chunk)
        def _zrow(i):
            for cc in range(ncc):
                sbuf[i, pl.ds(cc * _LANES, _LANES)] = zero

        nfull = zp // chunk
        for k in range(nfull):
            pltpu.sync_copy(sbuf, acc.at[pl.ds(s * zp + k * chunk, chunk)])
        rem = zp - nfull * chunk
        if rem:
            pltpu.sync_copy(sbuf.at[pl.ds(0, rem)],
                            acc.at[pl.ds(s * zp + nfull * chunk, rem)])

        plsc.subcore_barrier()

        cnt = base_cnt + jnp.where(wid < extra, 1, 0)

        @pl.loop(0, cnt)
        def _chunk(j):
            base = (wid + j * _NW) * chunk
            base8 = (wid + j * _NW) * (chunk // 8)
            pltpu.sync_copy(e_hbm.at[pl.ds(base8, chunk // 8)], ebuf)
            pltpu.sync_copy(pk_hbm.at[pl.ds(base8, chunk // 8)], pkbuf)
            pltpu.sync_copy(pid_hbm.at[pl.ds(base, chunk)], pidbuf)
            pltpu.sync_copy(x_hbm.at[pl.ds(base, chunk)], xbuf)

            @pl.loop(0, chunk // _LANES)
            def _pi(g):
                pv = pidbuf[pl.ds(g * _LANES, _LANES)]
                pid2buf[pl.ds(g * _LANES, _LANES)] = (
                    lax.shift_right_logical(pv, 3) + p)

            @pl.loop(0, chunk // 8)
            def _q(q):
                for rr in range(8):
                    i = q * 8 + rr
                    sl16 = pl.ds(rr * _LANES, _LANES)
                    eb = ebuf[q, sl16]
                    gv = jnp.bitwise_and(pkbuf[q, sl16], 7)
                    for cc in range(ncc):
                        sl = pl.ds(cc * _LANES, _LANES)
                        sbuf[i, sl] = xbuf[i, sl] * eb
                        dbuf[i, sl] = jnp.where(gv == cc, eb, 0.0)

            pltpu.sync_copy(sbuf, acc.at[pidbuf], add=True)
            pltpu.sync_copy(dbuf, acc.at[pid2buf], add=True)

        plsc.subcore_barrier()

        pltpu.sync_copy(acc.at[pl.ds(s * zp, zp)],
                        out_hbm.at[pl.ds(c * p2 + s * zp, zp)])

    return sc_kernel(x, e_packed, pid_packed, person_ids)


def _fin_body(ax_ref, ad_ref, pooled_ref, pid_ref):
    ax = ax_ref[...]
    ad = ad_ref[...]
    sx = ax[0] + ax[1]
    p = sx.shape[0]
    se = (ad[0] + ad[1])[:, 0:1]
    nonempty = se > 0.0
    inv = jnp.where(nonempty, 1.0 / jnp.where(nonempty, se, 1.0), 0.0)
    pooled_ref[...] = sx * inv
    lanes = jax.lax.broadcasted_iota(jnp.int32, (1, 1, p), 2)
    pid_ref[...] = jnp.where(nonempty.reshape(1, 1, p), lanes, _INT32_MAX)


def _fin_call(out_sc, p, p2, d):
    acc3 = out_sc.reshape(2, p2, d)
    den16 = acc3[:, p:p + p // 8].reshape(2, p, _LANES)
    return pl.pallas_call(
        _fin_body,
        grid=(1,),
        in_specs=[
            pl.BlockSpec((2, p, d), lambda k: (0, 0, 0)),
            pl.BlockSpec((2, p, _LANES), lambda k: (0, 0, 0)),
        ],
        out_specs=[
            pl.BlockSpec((p, d), lambda k: (0, 0)),
            pl.BlockSpec((1, 1, p), lambda k: (0, 0, 0)),
        ],
        out_shape=[
            jax.ShapeDtypeStruct((p, d), jnp.float32),
            jax.ShapeDtypeStruct((1, 1, p), jnp.int32),
        ],
    )(acc3, den16)


def kernel(x, W1, b1, W2, b2, person_ids):
    n, d = x.shape
    p = 10000
    p2 = 11264
    e_packed = _scores_call(x, W1, b1, W2, b2, block_rows=512)
    pid_packed = jnp.broadcast_to(
        person_ids.reshape(n // 8, 8, 1), (n // 8, 8, _LANES)
    ).reshape(n // 8, 128)
    out_sc = _sc_pool_call(x, e_packed, pid_packed, person_ids, p, p2,
                           chunk=64)
    pooled, pids3 = _fin_call(out_sc, p, p2, d)
    return pooled, pids3.reshape(p)


# speedup vs baseline: 4.9821x; 1.2485x over previous
import functools

import jax
import jax.numpy as jnp
from jax import lax
from jax.experimental import pallas as pl
from jax.experimental.pallas import tpu as pltpu
from jax.experimental.pallas import tpu_sc as plsc

_NC = 2
_NS = 16
_LANES = 16
_NW = _NC * _NS

_INT32_MAX = 2147483647


def _scores_body(x_ref, w1_ref, b1_ref, w2t_ref, b2_ref, e_ref):
    xb = x_ref[...]
    t = jnp.tanh(
        jnp.dot(xb, w1_ref[...], preferred_element_type=jnp.float32)
        + b1_ref[...]
    )
    s = jnp.sum(t * w2t_ref[...], axis=1, keepdims=True) + b2_ref[...]
    e = jnp.exp(s)
    r = e.shape[0]
    e8 = e.reshape(r // 8, 8, 1)
    e_ref[...] = jnp.broadcast_to(e8, (r // 8, 8, _LANES)).reshape(r // 8, 128)


def _scores_call(x, W1, b1, W2, b2, block_rows):
    n, d = x.shape
    h = W1.shape[1]
    nb = n // block_rows
    return pl.pallas_call(
        _scores_body,
        grid=(nb,),
        in_specs=[
            pl.BlockSpec((block_rows, d), lambda k: (k, 0)),
            pl.BlockSpec((d, h), lambda k: (0, 0)),
            pl.BlockSpec((1, h), lambda k: (0, 0)),
            pl.BlockSpec((1, h), lambda k: (0, 0)),
            pl.BlockSpec((1, 1), lambda k: (0, 0)),
        ],
        out_specs=pl.BlockSpec((block_rows // 8, 128), lambda k: (k, 0)),
        out_shape=jax.ShapeDtypeStruct((n // 8, 128), jnp.float32),
    )(x, W1, b1.reshape(1, h), W2.reshape(1, h), b2.reshape(1, 1))


def _sc_pool_call(x, e_packed, pid_packed, person_ids, p, p2, chunk):
    n, d = x.shape
    nchunks_total = n // chunk
    base_cnt = nchunks_total // _NW
    extra = nchunks_total - base_cnt * _NW
    ncc = d // _LANES
    zp = p2 // _NS

    mesh = plsc.VectorSubcoreMesh(
        core_axis_name="c", subcore_axis_name="s",
        num_cores=_NC, num_subcores=_NS,
    )

    @functools.partial(
        pl.kernel,
        out_type=jax.ShapeDtypeStruct((_NC * p2, d), jnp.float32),
        mesh=mesh,
        scratch_types=[
            pltpu.VMEM_SHARED((p2, d), jnp.float32),
            pltpu.VMEM((chunk, d), jnp.float32),
            pltpu.VMEM((chunk, d), jnp.float32),
            pltpu.VMEM((chunk, d), jnp.float32),
            pltpu.VMEM((chunk // 8, d), jnp.float32),
            pltpu.VMEM((chunk // 8, d), jnp.int32),
            pltpu.VMEM((chunk,), jnp.int32),
            pltpu.VMEM((chunk,), jnp.int32),
            pltpu.SemaphoreType.DMA,
            pltpu.SemaphoreType.DMA,
            pltpu.SemaphoreType.DMA,
            pltpu.SemaphoreType.DMA,
            pltpu.SemaphoreType.DMA,
            pltpu.SemaphoreType.DMA,
        ],
    )
    def sc_kernel(x_hbm, e_hbm, pk_hbm, pid_hbm, out_hbm,
                  acc, xbuf, sbuf, dbuf, ebuf, pkbuf, pidbuf, pid2buf,
                  sem_x, sem_e, sem_pk, sem_pid, sem_s, sem_d):
        c = lax.axis_index("c")
        s = lax.axis_index("s")
        wid = s * _NC + c
        zero = jnp.zeros((_LANES,), jnp.float32)

        @pl.loop(0, chunk)
        def _zrow(i):
            for cc in range(ncc):
                sbuf[i, pl.ds(cc * _LANES, _LANES)] = zero

        nfull = zp // chunk
        for k in range(nfull):
            pltpu.sync_copy(sbuf, acc.at[pl.ds(s * zp + k * chunk, chunk)])
        rem = zp - nfull * chunk
        if rem:
            pltpu.sync_copy(sbuf.at[pl.ds(0, rem)],
                            acc.at[pl.ds(s * zp + nfull * chunk, rem)])

        plsc.subcore_barrier()

        cnt = base_cnt + jnp.where(wid < extra, 1, 0)

        @pl.loop(0, cnt)
        def _chunk(j):
            base = (wid + j * _NW) * chunk
            base8 = (wid + j * _NW) * (chunk // 8)
            h_e = pltpu.async_copy(e_hbm.at[pl.ds(base8, chunk // 8)], ebuf,
                                   sem_e)
            h_pk = pltpu.async_copy(pk_hbm.at[pl.ds(base8, chunk // 8)],
                                    pkbuf, sem_pk)
            h_pid = pltpu.async_copy(pid_hbm.at[pl.ds(base, chunk)], pidbuf,
                                     sem_pid)
            h_x = pltpu.async_copy(x_hbm.at[pl.ds(base, chunk)], xbuf, sem_x)
            h_e.wait()
            h_pk.wait()
            h_pid.wait()
            h_x.wait()

            @pl.loop(0, chunk // _LANES)
            def _pi(g):
                pv = pidbuf[pl.ds(g * _LANES, _LANES)]
                pid2buf[pl.ds(g * _LANES, _LANES)] = (
                    lax.shift_right_logical(pv, 3) + p)

            @pl.loop(0, chunk // 8)
            def _q(q):
                for rr in range(8):
                    i = q * 8 + rr
                    sl16 = pl.ds(rr * _LANES, _LANES)
                    eb = ebuf[q, sl16]
                    gv = jnp.bitwise_and(pkbuf[q, sl16], 7)
                    for cc in range(ncc):
                        sl = pl.ds(cc * _LANES, _LANES)
                        sbuf[i, sl] = xbuf[i, sl] * eb
                        dbuf[i, sl] = jnp.where(gv == cc, eb, 0.0)

            h_s = pltpu.async_copy(sbuf, acc.at[pidbuf], sem_s, add=True)
            h_d = pltpu.async_copy(dbuf, acc.at[pid2buf], sem_d, add=True)
            h_s.wait()
            h_d.wait()

        plsc.subcore_barrier()

        pltpu.sync_copy(acc.at[pl.ds(s * zp, zp)],
                        out_hbm.at[pl.ds(c * p2 + s * zp, zp)])

    return sc_kernel(x, e_packed, pid_packed, person_ids)


def _fin_body(ax_ref, ad_ref, pooled_ref, pid_ref):
    ax = ax_ref[...]
    ad = ad_ref[...]
    sx = ax[0] + ax[1]
    p = sx.shape[0]
    se = (ad[0] + ad[1])[:, 0:1]
    nonempty = se > 0.0
    inv = jnp.where(nonempty, 1.0 / jnp.where(nonempty, se, 1.0), 0.0)
    pooled_ref[...] = sx * inv
    lanes = jax.lax.broadcasted_iota(jnp.int32, (1, 1, p), 2)
    pid_ref[...] = jnp.where(nonempty.reshape(1, 1, p), lanes, _INT32_MAX)


def _fin_call(out_sc, p, p2, d):
    acc3 = out_sc.reshape(2, p2, d)
    den16 = acc3[:, p:p + p // 8].reshape(2, p, _LANES)
    return pl.pallas_call(
        _fin_body,
        grid=(1,),
        in_specs=[
            pl.BlockSpec((2, p, d), lambda k: (0, 0, 0)),
            pl.BlockSpec((2, p, _LANES), lambda k: (0, 0, 0)),
        ],
        out_specs=[
            pl.BlockSpec((p, d), lambda k: (0, 0)),
            pl.BlockSpec((1, 1, p), lambda k: (0, 0, 0)),
        ],
        out_shape=[
            jax.ShapeDtypeStruct((p, d), jnp.float32),
            jax.ShapeDtypeStruct((1, 1, p), jnp.int32),
        ],
    )(acc3, den16)


def kernel(x, W1, b1, W2, b2, person_ids):
    n, d = x.shape
    p = 10000
    p2 = 11264
    e_packed = _scores_call(x, W1, b1, W2, b2, block_rows=512)
    pid_packed = jnp.broadcast_to(
        person_ids.reshape(n // 8, 8, 1), (n // 8, 8, _LANES)
    ).reshape(n // 8, 128)
    out_sc = _sc_pool_call(x, e_packed, pid_packed, person_ids, p, p2,
                           chunk=64)
    pooled, pids3 = _fin_call(out_sc, p, p2, d)
    return pooled, pids3.reshape(p)


# speedup vs baseline: 5.4177x; 1.0874x over previous
import functools

import jax
import jax.numpy as jnp
from jax import lax
from jax.experimental import pallas as pl
from jax.experimental.pallas import tpu as pltpu
from jax.experimental.pallas import tpu_sc as plsc

_NC = 2
_NS = 16
_LANES = 16
_NW = _NC * _NS

_INT32_MAX = 2147483647


def _scores_body(x_ref, w1_ref, b1_ref, w2t_ref, b2_ref, e_ref):
    xb = x_ref[...]
    t = jnp.tanh(
        jnp.dot(xb, w1_ref[...], preferred_element_type=jnp.float32)
        + b1_ref[...]
    )
    s = jnp.sum(t * w2t_ref[...], axis=1, keepdims=True) + b2_ref[...]
    e = jnp.exp(s)
    r = e.shape[0]
    e8 = e.reshape(r // 8, 8, 1)
    e_ref[...] = jnp.broadcast_to(e8, (r // 8, 8, _LANES)).reshape(r // 8, 128)


def _scores_call(x, W1, b1, W2, b2, block_rows):
    n, d = x.shape
    h = W1.shape[1]
    nb = n // block_rows
    return pl.pallas_call(
        _scores_body,
        grid=(nb,),
        in_specs=[
            pl.BlockSpec((block_rows, d), lambda k: (k, 0)),
            pl.BlockSpec((d, h), lambda k: (0, 0)),
            pl.BlockSpec((1, h), lambda k: (0, 0)),
            pl.BlockSpec((1, h), lambda k: (0, 0)),
            pl.BlockSpec((1, 1), lambda k: (0, 0)),
        ],
        out_specs=pl.BlockSpec((block_rows // 8, 128), lambda k: (k, 0)),
        out_shape=jax.ShapeDtypeStruct((n // 8, 128), jnp.float32),
    )(x, W1, b1.reshape(1, h), W2.reshape(1, h), b2.reshape(1, 1))


def _sc_pool_call(x, e_packed, pid_packed, person_ids, p, p2, chunk):
    n, d = x.shape
    nchunks_total = n // chunk
    base_cnt = nchunks_total // _NW
    extra = nchunks_total - base_cnt * _NW
    ncc = d // _LANES
    zp = p2 // _NS

    mesh = plsc.VectorSubcoreMesh(
        core_axis_name="c", subcore_axis_name="s",
        num_cores=_NC, num_subcores=_NS,
    )

    @functools.partial(
        pl.kernel,
        out_type=jax.ShapeDtypeStruct((_NC * p2, d), jnp.float32),
        mesh=mesh,
        scratch_types=[
            pltpu.VMEM_SHARED((p2, d), jnp.float32),
            pltpu.VMEM((chunk, d), jnp.float32),
            pltpu.VMEM((chunk, d), jnp.float32),
            pltpu.VMEM((chunk, d), jnp.float32),
            pltpu.VMEM((chunk // 8, d), jnp.float32),
            pltpu.VMEM((chunk // 8, d), jnp.int32),
            pltpu.VMEM((chunk,), jnp.int32),
            pltpu.VMEM((chunk,), jnp.int32),
            pltpu.VMEM((chunk,), jnp.int32),
            pltpu.SemaphoreType.DMA,
            pltpu.SemaphoreType.DMA,
            pltpu.SemaphoreType.DMA,
            pltpu.SemaphoreType.DMA,
            pltpu.SemaphoreType.DMA,
            pltpu.SemaphoreType.DMA,
        ],
    )
    def sc_kernel(x_hbm, e_hbm, pk_hbm, pid_hbm, out_hbm,
                  acc, xbuf, sbuf, dbuf, ebuf, pkbuf, pidbuf, pid2buf,
                  pid3buf, sem_x, sem_e, sem_pk, sem_pid, sem_s, sem_d):
        c = lax.axis_index("c")
        s = lax.axis_index("s")
        wid = s * _NC + c
        zero = jnp.zeros((_LANES,), jnp.float32)

        @pl.loop(0, chunk)
        def _zrow(i):
            for cc in range(ncc):
                sbuf[i, pl.ds(cc * _LANES, _LANES)] = zero

        nfull = zp // chunk
        for k in range(nfull):
            pltpu.sync_copy(sbuf, acc.at[pl.ds(s * zp + k * chunk, chunk)])
        rem = zp - nfull * chunk
        if rem:
            pltpu.sync_copy(sbuf.at[pl.ds(0, rem)],
                            acc.at[pl.ds(s * zp + nfull * chunk, rem)])

        plsc.subcore_barrier()

        cnt = base_cnt + jnp.where(wid < extra, 1, 0)

        @pl.loop(0, cnt)
        def _chunk(j):
            base = (wid + j * _NW) * chunk
            base8 = (wid + j * _NW) * (chunk // 8)
            h_e = pltpu.async_copy(e_hbm.at[pl.ds(base8, chunk // 8)], ebuf,
                                   sem_e)
            h_pk = pltpu.async_copy(pk_hbm.at[pl.ds(base8, chunk // 8)],
                                    pkbuf, sem_pk)
            h_pid = pltpu.async_copy(pid_hbm.at[pl.ds(base, chunk)], pidbuf,
                                     sem_pid)
            h_x = pltpu.async_copy(x_hbm.at[pl.ds(base, chunk)], xbuf, sem_x)

            @pl.when(j > 0)
            def _drain():
                pltpu.make_async_copy(sbuf, acc.at[pidbuf], sem_s).wait()
                pltpu.make_async_copy(dbuf, acc.at[pid2buf], sem_d).wait()

            h_e.wait()
            h_pk.wait()
            h_pid.wait()

            @pl.loop(0, chunk // _LANES)
            def _pi(g):
                slg = pl.ds(g * _LANES, _LANES)
                pv = pidbuf[slg]
                pid2buf[slg] = lax.shift_right_logical(pv, 3) + p
                pid3buf[slg] = pv

            @pl.loop(0, chunk // 8)
            def _qd(q):
                for rr in range(8):
                    i = q * 8 + rr
                    sl16 = pl.ds(rr * _LANES, _LANES)
                    eb = ebuf[q, sl16]
                    gv = jnp.bitwise_and(pkbuf[q, sl16], 7)
                    for cc in range(ncc):
                        sl = pl.ds(cc * _LANES, _LANES)
                        dbuf[i, sl] = jnp.where(gv == cc, eb, 0.0)

            h_x.wait()

            @pl.loop(0, chunk // 8)
            def _q(q):
                for rr in range(8):
                    i = q * 8 + rr
                    sl16 = pl.ds(rr * _LANES, _LANES)
                    eb = ebuf[q, sl16]
                    for cc in range(ncc):
                        sl = pl.ds(cc * _LANES, _LANES)
                        sbuf[i, sl] = xbuf[i, sl] * eb

            pltpu.async_copy(sbuf, acc.at[pid3buf], sem_s, add=True)
            pltpu.async_copy(dbuf, acc.at[pid2buf], sem_d, add=True)

        pltpu.make_async_copy(sbuf, acc.at[pid3buf], sem_s).wait()
        pltpu.make_async_copy(dbuf, acc.at[pid2buf], sem_d).wait()

        plsc.subcore_barrier()

        pltpu.sync_copy(acc.at[pl.ds(s * zp, zp)],
                        out_hbm.at[pl.ds(c * p2 + s * zp, zp)])

    return sc_kernel(x, e_packed, pid_packed, person_ids)


def _fin_body(ax_ref, ad_ref, pooled_ref, pid_ref):
    ax = ax_ref[...]
    ad = ad_ref[...]
    sx = ax[0] + ax[1]
    p = sx.shape[0]
    se = (ad[0] + ad[1])[:, 0:1]
    nonempty = se > 0.0
    inv = jnp.where(nonempty, 1.0 / jnp.where(nonempty, se, 1.0), 0.0)
    pooled_ref[...] = sx * inv
    lanes = jax.lax.broadcasted_iota(jnp.int32, (1, 1, p), 2)
    pid_ref[...] = jnp.where(nonempty.reshape(1, 1, p), lanes, _INT32_MAX)


def _fin_call(out_sc, p, p2, d):
    acc3 = out_sc.reshape(2, p2, d)
    den16 = acc3[:, p:p + p // 8].reshape(2, p, _LANES)
    return pl.pallas_call(
        _fin_body,
        grid=(1,),
        in_specs=[
            pl.BlockSpec((2, p, d), lambda k: (0, 0, 0)),
            pl.BlockSpec((2, p, _LANES), lambda k: (0, 0, 0)),
        ],
        out_specs=[
            pl.BlockSpec((p, d), lambda k: (0, 0)),
            pl.BlockSpec((1, 1, p), lambda k: (0, 0, 0)),
        ],
        out_shape=[
            jax.ShapeDtypeStruct((p, d), jnp.float32),
            jax.ShapeDtypeStruct((1, 1, p), jnp.int32),
        ],
    )(acc3, den16)


def kernel(x, W1, b1, W2, b2, person_ids):
    n, d = x.shape
    p = 10000
    p2 = 11264
    e_packed = _scores_call(x, W1, b1, W2, b2, block_rows=512)
    pid_packed = jnp.broadcast_to(
        person_ids.reshape(n // 8, 8, 1), (n // 8, 8, _LANES)
    ).reshape(n // 8, 128)
    out_sc = _sc_pool_call(x, e_packed, pid_packed, person_ids, p, p2,
                           chunk=64)
    pooled, pids3 = _fin_call(out_sc, p, p2, d)
    return pooled, pids3.reshape(p)
